# Initial kernel scaffold; baseline (speedup 1.0000x reference)
import jax
import jax.numpy as jnp
from jax.experimental import pallas as pl


def kernel(subs, rels, entity_embed, relation_embed, hop_embed, W1, b1, W2, b2, w_n, noise_eps):
    raise NotImplementedError("write your pallas kernel here")


# speedup vs baseline: 2.7756x; 2.7756x over previous
import jax
import jax.numpy as jnp
from jax import lax
from jax.experimental import pallas as pl
from jax.experimental.pallas import tpu as pltpu
from jax.experimental.pallas import tpu_sc as plsc

B = 16384
HID = 1024
HOPS = 8
NEXP = 4

_NC = 2
_NS = 16
_NW = _NC * _NS
_RPW = B // _NW
_CH = 64
_NCHUNK = _RPW // _CH

_R = 512
_NSTEP = B // _R


def _sc_gather_body(ent_hbm, rel_hbm, subs_hbm, rels_hbm, out_sub, out_rel,
                    idx_v, rows_v, sem):
    wid = lax.axis_index("s") * _NC + lax.axis_index("c")
    base = wid * _RPW
    for tab, ind, out in ((ent_hbm, subs_hbm, out_sub),
                          (rel_hbm, rels_hbm, out_rel)):
        for c in range(_NCHUNK):
            off = base + c * _CH
            pltpu.sync_copy(ind.at[pl.ds(off, _CH)], idx_v)
            pltpu.async_copy(tab.at[idx_v], rows_v, sem).wait()
            pltpu.sync_copy(rows_v, out.at[pl.ds(off, _CH)])


def _gather_rows(entity_embed, relation_embed, subs, rels):
    mesh = plsc.VectorSubcoreMesh(core_axis_name="c", subcore_axis_name="s",
                                  num_cores=_NC, num_subcores=_NS)
    return pl.kernel(
        _sc_gather_body,
        out_type=(jax.ShapeDtypeStruct((B, HID), jnp.float32),
                  jax.ShapeDtypeStruct((B, HID), jnp.float32)),
        mesh=mesh,
        scratch_types=(pltpu.VMEM((_CH,), jnp.int32),
                       pltpu.VMEM((_CH, HID), jnp.float32),
                       pltpu.SemaphoreType.DMA),
    )(entity_embed, relation_embed, subs, rels)


def _tc_body(sub_ref, rel_ref, W1_ref, b1_ref, W2_ref, b2_ref, hop_ref,
             wn_ref, noise_ref, G_ref, Q_ref, acc_ref):
    i = pl.program_id(0)

    @pl.when(i == 0)
    def _():
        acc_ref[...] = jnp.zeros_like(acc_ref)

    dn = (((1,), (0,)), ((), ()))
    z = lax.dot_general(sub_ref[...], W1_ref[0:HID, :], dn,
                        preferred_element_type=jnp.float32)
    z = z + lax.dot_general(rel_ref[...], W1_ref[HID:2 * HID, :], dn,
                            preferred_element_type=jnp.float32)
    z = z + b1_ref[...]
    h = jnp.maximum(z, 0.0)
    acc_ref[...] += jnp.sum(h, axis=0, keepdims=True)

    @pl.when(i == _NSTEP - 1)
    def _():
        c_i = acc_ref[...] * (1.0 / B)
        c_i = lax.dot_general(c_i, W2_ref[...], dn,
                              preferred_element_type=jnp.float32) + b2_ref[...]
        q = lax.dot_general(c_i, hop_ref[...], (((1,), (1,)), ((), ())),
                            preferred_element_type=jnp.float32)
        sx = jnp.sum(c_i * wn_ref[...])
        sigma = jnp.maximum(sx, 0.0) + jnp.log1p(jnp.exp(-jnp.abs(sx)))
        q = q + noise_ref[...] * sigma

        iot = lax.broadcasted_iota(jnp.int32, (1, HOPS), 1)
        rank = jnp.zeros((1, HOPS), jnp.int32)
        for j in range(HOPS):
            qj = q[0, j]
            beats = (qj > q) | ((qj == q) & (j < iot))
            rank = rank + beats.astype(jnp.int32)
        sel = rank < NEXP
        m = jnp.max(jnp.where(sel, q, -1e30))
        e = jnp.where(sel, jnp.exp(q - m), 0.0)
        G_ref[...] = e / jnp.sum(e)
        Q_ref[...] = q


def _moe_head(sub_rows, rel_rows, W1, b1, W2, b2, hop_embed, wn_row, noise_row):
    g, q = pl.pallas_call(
        _tc_body,
        grid=(_NSTEP,),
        in_specs=[
            pl.BlockSpec((_R, HID), lambda i: (i, 0)),
            pl.BlockSpec((_R, HID), lambda i: (i, 0)),
            pl.BlockSpec((2 * HID, HID), lambda i: (0, 0)),
            pl.BlockSpec((1, HID), lambda i: (0, 0)),
            pl.BlockSpec((HID, HID), lambda i: (0, 0)),
            pl.BlockSpec((1, HID), lambda i: (0, 0)),
            pl.BlockSpec((HOPS, HID), lambda i: (0, 0)),
            pl.BlockSpec((1, HID), lambda i: (0, 0)),
            pl.BlockSpec((1, HOPS), lambda i: (0, 0)),
        ],
        out_specs=[
            pl.BlockSpec((1, HOPS), lambda i: (0, 0)),
            pl.BlockSpec((1, HOPS), lambda i: (0, 0)),
        ],
        out_shape=[
            jax.ShapeDtypeStruct((1, HOPS), jnp.float32),
            jax.ShapeDtypeStruct((1, HOPS), jnp.float32),
        ],
        scratch_shapes=[pltpu.VMEM((1, HID), jnp.float32)],
        compiler_params=pltpu.CompilerParams(
            dimension_semantics=("arbitrary",)),
    )(sub_rows, rel_rows, W1, b1, W2, b2, hop_embed, wn_row, noise_row)
    return g, q


def kernel(subs, rels, entity_embed, relation_embed, hop_embed, W1, b1, W2,
           b2, w_n, noise_eps):
    sub_rows, rel_rows = _gather_rows(entity_embed, relation_embed, subs, rels)
    g, q = _moe_head(sub_rows, rel_rows, W1,
                     b1.reshape(1, HID), W2, b2.reshape(1, HID),
                     hop_embed, w_n.reshape(1, HID),
                     noise_eps.reshape(1, HOPS))
    return (g.reshape(HOPS), q.reshape(HOPS))


# speedup vs baseline: 2.7870x; 1.0041x over previous
import jax
import jax.numpy as jnp
from jax import lax
from jax.experimental import pallas as pl
from jax.experimental.pallas import tpu as pltpu
from jax.experimental.pallas import tpu_sc as plsc

B = 16384
HID = 1024
HOPS = 8
NEXP = 4

_NC = 2
_NS = 16
_NW = _NC * _NS
_RPW = B // _NW
_CH = 64
_NCHUNK = _RPW // _CH

_R = 512
_NSTEP = B // _R


def _sc_gather_body(ent_hbm, rel_hbm, subs_hbm, rels_hbm, out_sub, out_rel,
                    idx_v, rows_v, sem):
    wid = lax.axis_index("s") * _NC + lax.axis_index("c")
    base = wid * _RPW
    for tab, ind, out in ((ent_hbm, subs_hbm, out_sub),
                          (rel_hbm, rels_hbm, out_rel)):
        for c in range(_NCHUNK):
            off = base + c * _CH
            pltpu.sync_copy(ind.at[pl.ds(off, _CH)], idx_v)
            pltpu.async_copy(tab.at[idx_v], rows_v, sem).wait()
            pltpu.sync_copy(rows_v, out.at[pl.ds(off, _CH)])


def _gather_rows(entity_embed, relation_embed, subs, rels):
    mesh = plsc.VectorSubcoreMesh(core_axis_name="c", subcore_axis_name="s",
                                  num_cores=_NC, num_subcores=_NS)
    return pl.kernel(
        _sc_gather_body,
        out_type=(jax.ShapeDtypeStruct((B, HID), jnp.float32),
                  jax.ShapeDtypeStruct((B, HID), jnp.float32)),
        mesh=mesh,
        scratch_types=(pltpu.VMEM((_CH,), jnp.int32),
                       pltpu.VMEM((_CH, HID), jnp.float32),
                       pltpu.SemaphoreType.DMA),
    )(entity_embed, relation_embed, subs, rels)


def _tc_body(sub_ref, rel_ref, W1_ref, b1_ref, W2_ref, b2_ref, hop_ref,
             wn_ref, noise_ref, G_ref, Q_ref, acc_ref):
    i = pl.program_id(0)

    @pl.when(i == 0)
    def _():
        acc_ref[...] = jnp.zeros_like(acc_ref)

    dn = (((1,), (0,)), ((), ()))
    z = lax.dot_general(sub_ref[...].astype(jnp.bfloat16), W1_ref[0:HID, :],
                        dn, preferred_element_type=jnp.float32)
    z = z + lax.dot_general(rel_ref[...].astype(jnp.bfloat16),
                            W1_ref[HID:2 * HID, :], dn,
                            preferred_element_type=jnp.float32)
    z = z + b1_ref[...]
    h = jnp.maximum(z, 0.0)
    acc_ref[...] += jnp.sum(h, axis=0, keepdims=True)

    @pl.when(i == _NSTEP - 1)
    def _():
        c_i = acc_ref[...] * (1.0 / B)
        c_i = lax.dot_general(c_i, W2_ref[...], dn,
                              preferred_element_type=jnp.float32) + b2_ref[...]
        q = lax.dot_general(c_i, hop_ref[...], (((1,), (1,)), ((), ())),
                            preferred_element_type=jnp.float32)
        sx = jnp.sum(c_i * wn_ref[...])
        sigma = jnp.maximum(sx, 0.0) + jnp.log1p(jnp.exp(-jnp.abs(sx)))
        q = q + noise_ref[...] * sigma

        iot = lax.broadcasted_iota(jnp.int32, (1, HOPS), 1)
        rank = jnp.zeros((1, HOPS), jnp.int32)
        for j in range(HOPS):
            qj = q[0, j]
            beats = (qj > q) | ((qj == q) & (j < iot))
            rank = rank + beats.astype(jnp.int32)
        sel = rank < NEXP
        m = jnp.max(jnp.where(sel, q, -1e30))
        e = jnp.where(sel, jnp.exp(q - m), 0.0)
        G_ref[...] = e / jnp.sum(e)
        Q_ref[...] = q


def _moe_head(sub_rows, rel_rows, W1, b1, W2, b2, hop_embed, wn_row, noise_row):
    g, q = pl.pallas_call(
        _tc_body,
        grid=(_NSTEP,),
        in_specs=[
            pl.BlockSpec((_R, HID), lambda i: (i, 0)),
            pl.BlockSpec((_R, HID), lambda i: (i, 0)),
            pl.BlockSpec((2 * HID, HID), lambda i: (0, 0)),
            pl.BlockSpec((1, HID), lambda i: (0, 0)),
            pl.BlockSpec((HID, HID), lambda i: (0, 0)),
            pl.BlockSpec((1, HID), lambda i: (0, 0)),
            pl.BlockSpec((HOPS, HID), lambda i: (0, 0)),
            pl.BlockSpec((1, HID), lambda i: (0, 0)),
            pl.BlockSpec((1, HOPS), lambda i: (0, 0)),
        ],
        out_specs=[
            pl.BlockSpec((1, HOPS), lambda i: (0, 0)),
            pl.BlockSpec((1, HOPS), lambda i: (0, 0)),
        ],
        out_shape=[
            jax.ShapeDtypeStruct((1, HOPS), jnp.float32),
            jax.ShapeDtypeStruct((1, HOPS), jnp.float32),
        ],
        scratch_shapes=[pltpu.VMEM((1, HID), jnp.float32)],
        compiler_params=pltpu.CompilerParams(
            dimension_semantics=("arbitrary",)),
    )(sub_rows, rel_rows, W1, b1, W2, b2, hop_embed, wn_row, noise_row)
    return g, q


def kernel(subs, rels, entity_embed, relation_embed, hop_embed, W1, b1, W2,
           b2, w_n, noise_eps):
    sub_rows, rel_rows = _gather_rows(entity_embed, relation_embed, subs, rels)
    g, q = _moe_head(sub_rows, rel_rows, W1.astype(jnp.bfloat16),
                     b1.reshape(1, HID), W2, b2.reshape(1, HID),
                     hop_embed, w_n.reshape(1, HID),
                     noise_eps.reshape(1, HOPS))
    return (g.reshape(HOPS), q.reshape(HOPS))
